# baseline (speedup 1.0000x reference)
import functools

import jax
import jax.numpy as jnp
from jax.experimental import pallas as pl

_TOKEN_TILE = 128


def _rpq_kernel(x_ref, w_ref, cb_ref, out_ref):
    x = x_ref[...]
    w = w_ref[...]
    z = jax.lax.dot_general(
        x.astype(jnp.bfloat16), w.astype(jnp.bfloat16),
        (((1,), (1,)), ((), ())),
        preferred_element_type=jnp.float32,
    )
    mu = jnp.mean(z, axis=-1, keepdims=True)
    var = jnp.mean((z - mu) * (z - mu), axis=-1, keepdims=True)
    z = (z - mu) / jnp.sqrt(var + 1e-5)

    cb = cb_ref[...]
    c2 = jnp.sum(cb * cb, axis=-1, keepdims=True)
    t = z.shape[0]
    z_aug = jnp.concatenate(
        [(-2.0 * z).astype(jnp.bfloat16), jnp.ones((t, 1), jnp.bfloat16)], axis=1)
    cb_aug = jnp.concatenate([cb.astype(jnp.bfloat16), c2.astype(jnp.bfloat16)], axis=1)
    d = jax.lax.dot_general(
        z_aug, cb_aug, (((1,), (1,)), ((), ())),
        preferred_element_type=jnp.float32,
    )
    out_ref[...] = jnp.argmin(d, axis=-1).astype(jnp.int32)


@functools.partial(jax.jit, static_argnames=())
def kernel(x, W, codebook):
    B, L, D = x.shape
    K, CD = codebook.shape
    n = B * L
    xf = x.reshape(n, D)
    tile = _TOKEN_TILE
    grid = (n // tile,)
    codes = pl.pallas_call(
        _rpq_kernel,
        grid=grid,
        in_specs=[
            pl.BlockSpec((tile, D), lambda i: (i, 0)),
            pl.BlockSpec((CD, D), lambda i: (0, 0)),
            pl.BlockSpec((K, CD), lambda i: (0, 0)),
        ],
        out_specs=pl.BlockSpec((tile,), lambda i: (i,)),
        out_shape=jax.ShapeDtypeStruct((n,), jnp.int32),
    )(xf, W, codebook)
    return codes.reshape(B, L)


# speedup vs baseline: 1.7535x; 1.7535x over previous
import functools

import jax
import jax.numpy as jnp
from jax.experimental import pallas as pl

_TOKEN_TILE = 256


def _rpq_kernel(x_ref, w_ref, cb_ref, out_ref):
    x = x_ref[...]
    w = w_ref[...]
    z = jax.lax.dot_general(
        x.astype(jnp.bfloat16), w.astype(jnp.bfloat16),
        (((1,), (1,)), ((), ())),
        preferred_element_type=jnp.float32,
    )
    mu = jnp.mean(z, axis=-1, keepdims=True)
    var = jnp.mean((z - mu) * (z - mu), axis=-1, keepdims=True)
    z = (z - mu) / jnp.sqrt(var + 1e-5)

    cb = cb_ref[...]
    c2 = jnp.sum(cb * cb, axis=-1, keepdims=True)
    t = z.shape[0]
    z_aug = jnp.concatenate(
        [(-2.0 * z).astype(jnp.bfloat16), jnp.ones((t, 1), jnp.bfloat16)], axis=1)
    cb_aug = jnp.concatenate([cb.astype(jnp.bfloat16), c2.astype(jnp.bfloat16)], axis=1)
    d = jax.lax.dot_general(
        z_aug, cb_aug, (((1,), (1,)), ((), ())),
        preferred_element_type=jnp.float32,
    )
    out_ref[...] = jnp.argmin(d, axis=-1).astype(jnp.int32)


@functools.partial(jax.jit, static_argnames=())
def kernel(x, W, codebook):
    B, L, D = x.shape
    K, CD = codebook.shape
    n = B * L
    xf = x.reshape(n, D)
    tile = _TOKEN_TILE
    grid = (n // tile,)
    codes = pl.pallas_call(
        _rpq_kernel,
        grid=grid,
        in_specs=[
            pl.BlockSpec((tile, D), lambda i: (i, 0)),
            pl.BlockSpec((CD, D), lambda i: (0, 0)),
            pl.BlockSpec((K, CD), lambda i: (0, 0)),
        ],
        out_specs=pl.BlockSpec((tile,), lambda i: (i,)),
        out_shape=jax.ShapeDtypeStruct((n,), jnp.int32),
    )(xf, W, codebook)
    return codes.reshape(B, L)
